# Initial kernel scaffold; baseline (speedup 1.0000x reference)
import jax
import jax.numpy as jnp
from jax.experimental import pallas as pl


def kernel(lut, x):
    raise NotImplementedError("write your pallas kernel here")


# speedup vs baseline: 138.3309x; 138.3309x over previous
import functools

import jax
import jax.numpy as jnp
from jax import lax
from jax.experimental import pallas as pl
from jax.experimental.pallas import tpu as pltpu
from jax.experimental.pallas import tpu_sc as plsc

_DIM = 33
_NLUT = 3 * _DIM ** 3
_CSTRIDE = _DIM ** 3
_PLANE = 512 * 512
_B = 16
_PPW = _B * _PLANE // 32
_CHUNK = 1024
_NCHUNK = _PPW // _CHUNK
_GROUPS = _CHUNK // 16
_SCALE = float(_DIM - 1)


def _interp_group(lut_v, xin, xout, o):
    xr = xin[pl.ds(o, 16)]
    xg = xin[pl.ds(_CHUNK + o, 16)]
    xb = xin[pl.ds(2 * _CHUNK + o, 16)]
    r = xr * _SCALE
    g = xg * _SCALE
    b = xb * _SCALE
    r0 = jnp.minimum(r.astype(jnp.int32), _DIM - 2)
    g0 = jnp.minimum(g.astype(jnp.int32), _DIM - 2)
    b0 = jnp.minimum(b.astype(jnp.int32), _DIM - 2)
    fr = r - r0.astype(jnp.float32)
    fg = g - g0.astype(jnp.float32)
    fb = b - b0.astype(jnp.float32)
    base = b0 * (_DIM * _DIM) + g0 * _DIM + r0
    for c in range(3):
        bc = base + c * _CSTRIDE
        v000 = plsc.load_gather(lut_v, [bc])
        v001 = plsc.load_gather(lut_v, [bc + 1])
        v010 = plsc.load_gather(lut_v, [bc + _DIM])
        v011 = plsc.load_gather(lut_v, [bc + (_DIM + 1)])
        v100 = plsc.load_gather(lut_v, [bc + _DIM * _DIM])
        v101 = plsc.load_gather(lut_v, [bc + (_DIM * _DIM + 1)])
        v110 = plsc.load_gather(lut_v, [bc + (_DIM * _DIM + _DIM)])
        v111 = plsc.load_gather(lut_v, [bc + (_DIM * _DIM + _DIM + 1)])
        a00 = v000 + fr * (v001 - v000)
        a01 = v010 + fr * (v011 - v010)
        a10 = v100 + fr * (v101 - v100)
        a11 = v110 + fr * (v111 - v110)
        e0 = a00 + fg * (a01 - a00)
        e1 = a10 + fg * (a11 - a10)
        xout[pl.ds(c * _CHUNK + o, 16)] = e0 + fb * (e1 - e0)


def _body(lut_hbm, x_hbm, out_hbm, lut_v, xin, xout):
    img = lax.axis_index("s")
    half = lax.axis_index("c")
    base = half * _PPW
    pltpu.sync_copy(lut_hbm, lut_v)

    def chunk_body(gk, carry):
        off = base + gk * _CHUNK
        for c in range(3):
            pltpu.sync_copy(
                x_hbm.at[pl.ds((img * 3 + c) * _PLANE + off, _CHUNK)],
                xin.at[pl.ds(c * _CHUNK, _CHUNK)],
            )

        def grp(j, c2):
            _interp_group(lut_v, xin, xout, j * 16)
            return c2

        lax.fori_loop(0, _GROUPS, grp, 0)
        for c in range(3):
            pltpu.sync_copy(
                xout.at[pl.ds(c * _CHUNK, _CHUNK)],
                out_hbm.at[pl.ds((img * 3 + c) * _PLANE + off, _CHUNK)],
            )
        return carry

    lax.fori_loop(0, _NCHUNK, chunk_body, 0)


@functools.cache
def _build():
    mesh = plsc.VectorSubcoreMesh(
        core_axis_name="c", subcore_axis_name="s", num_cores=2, num_subcores=16
    )
    return pl.kernel(
        _body,
        out_type=jax.ShapeDtypeStruct((_B * 3 * _PLANE,), jnp.float32),
        mesh=mesh,
        scratch_types=[
            pltpu.VMEM((_NLUT,), jnp.float32),
            pltpu.VMEM((3 * _CHUNK,), jnp.float32),
            pltpu.VMEM((3 * _CHUNK,), jnp.float32),
        ],
        compiler_params=pltpu.CompilerParams(needs_layout_passes=False),
    )


@jax.jit
def kernel(lut, x):
    lut_flat = lut.reshape(_NLUT)
    x_flat = x.reshape(_B * 3 * _PLANE)
    out = _build()(lut_flat, x_flat)
    return out.reshape(_B, 3, 512, 512)


# speedup vs baseline: 191.5488x; 1.3847x over previous
import functools

import jax
import jax.numpy as jnp
from jax import lax
from jax.experimental import pallas as pl
from jax.experimental.pallas import tpu as pltpu
from jax.experimental.pallas import tpu_sc as plsc

_DIM = 33
_NLUT = 3 * _DIM ** 3
_CSTRIDE = _DIM ** 3
_PLANE = 512 * 512
_B = 16
_PPW = _B * _PLANE // 32
_CHUNK = 1024
_NCHUNK = _PPW // _CHUNK
_GROUPS = _CHUNK // 16
_SCALE = float(_DIM - 1)


def _interp_group(lut_v, xin, xout, o):
    xr = xin[pl.ds(o, 16)]
    xg = xin[pl.ds(_CHUNK + o, 16)]
    xb = xin[pl.ds(2 * _CHUNK + o, 16)]
    r = xr * _SCALE
    g = xg * _SCALE
    b = xb * _SCALE
    r0 = jnp.minimum(r.astype(jnp.int32), _DIM - 2)
    g0 = jnp.minimum(g.astype(jnp.int32), _DIM - 2)
    b0 = jnp.minimum(b.astype(jnp.int32), _DIM - 2)
    fr = r - r0.astype(jnp.float32)
    fg = g - g0.astype(jnp.float32)
    fb = b - b0.astype(jnp.float32)
    base = b0 * (_DIM * _DIM) + g0 * _DIM + r0
    for c in range(3):
        bc = base + c * _CSTRIDE
        v000 = plsc.load_gather(lut_v, [bc])
        v001 = plsc.load_gather(lut_v, [bc + 1])
        v010 = plsc.load_gather(lut_v, [bc + _DIM])
        v011 = plsc.load_gather(lut_v, [bc + (_DIM + 1)])
        v100 = plsc.load_gather(lut_v, [bc + _DIM * _DIM])
        v101 = plsc.load_gather(lut_v, [bc + (_DIM * _DIM + 1)])
        v110 = plsc.load_gather(lut_v, [bc + (_DIM * _DIM + _DIM)])
        v111 = plsc.load_gather(lut_v, [bc + (_DIM * _DIM + _DIM + 1)])
        a00 = v000 + fr * (v001 - v000)
        a01 = v010 + fr * (v011 - v010)
        a10 = v100 + fr * (v101 - v100)
        a11 = v110 + fr * (v111 - v110)
        e0 = a00 + fg * (a01 - a00)
        e1 = a10 + fg * (a11 - a10)
        xout[pl.ds(c * _CHUNK + o, 16)] = e0 + fb * (e1 - e0)


def _body(lut_hbm, x_hbm, out_hbm, lut_v, xin0, xin1, xout0, xout1,
          si0, si1, so0, so1):
    img = lax.axis_index("s")
    half = lax.axis_index("c")
    base = half * _PPW
    pltpu.sync_copy(lut_hbm, lut_v)

    def in_copy(g, buf, sem):
        off = base + g * _CHUNK
        return [
            pltpu.make_async_copy(
                x_hbm.at[pl.ds((img * 3 + c) * _PLANE + off, _CHUNK)],
                buf.at[pl.ds(c * _CHUNK, _CHUNK)],
                sem,
            )
            for c in range(3)
        ]

    def out_copy(g, buf, sem):
        off = base + g * _CHUNK
        return [
            pltpu.make_async_copy(
                buf.at[pl.ds(c * _CHUNK, _CHUNK)],
                out_hbm.at[pl.ds((img * 3 + c) * _PLANE + off, _CHUNK)],
                sem,
            )
            for c in range(3)
        ]

    def compute(xin, xout):
        def grp(j, c2):
            _interp_group(lut_v, xin, xout, j * 16)
            return c2

        lax.fori_loop(0, _GROUPS, grp, 0)

    for d in in_copy(0, xin0, si0):
        d.start()
    for d in in_copy(1, xin1, si1):
        d.start()

    def pair(i, carry):
        g0 = 2 * i
        for buf_i, (g, xin, xout, si, so) in enumerate(
            ((g0, xin0, xout0, si0, so0), (g0 + 1, xin1, xout1, si1, so1))
        ):
            for d in in_copy(g, xin, si):
                d.wait()

            @pl.when(i > 0)
            def _():
                for d in out_copy(g, xout, so):
                    d.wait()

            compute(xin, xout)
            for d in out_copy(g, xout, so):
                d.start()

            @pl.when(g + 2 < _NCHUNK)
            def _():
                for d in in_copy(g + 2, xin, si):
                    d.start()

        return carry

    lax.fori_loop(0, _NCHUNK // 2, pair, 0)
    for d in out_copy(_NCHUNK - 2, xout0, so0):
        d.wait()
    for d in out_copy(_NCHUNK - 1, xout1, so1):
        d.wait()


@functools.cache
def _build():
    mesh = plsc.VectorSubcoreMesh(
        core_axis_name="c", subcore_axis_name="s", num_cores=2, num_subcores=16
    )
    return pl.kernel(
        _body,
        out_type=jax.ShapeDtypeStruct((_B * 3 * _PLANE,), jnp.float32),
        mesh=mesh,
        scratch_types=[
            pltpu.VMEM((_NLUT,), jnp.float32),
            pltpu.VMEM((3 * _CHUNK,), jnp.float32),
            pltpu.VMEM((3 * _CHUNK,), jnp.float32),
            pltpu.VMEM((3 * _CHUNK,), jnp.float32),
            pltpu.VMEM((3 * _CHUNK,), jnp.float32),
            pltpu.SemaphoreType.DMA,
            pltpu.SemaphoreType.DMA,
            pltpu.SemaphoreType.DMA,
            pltpu.SemaphoreType.DMA,
        ],
        compiler_params=pltpu.CompilerParams(needs_layout_passes=False),
    )


@jax.jit
def kernel(lut, x):
    lut_flat = lut.reshape(_NLUT)
    x_flat = x.reshape(_B * 3 * _PLANE)
    out = _build()(lut_flat, x_flat)
    return out.reshape(_B, 3, 512, 512)


# speedup vs baseline: 343.6905x; 1.7943x over previous
import functools

import jax
import jax.numpy as jnp
from jax import lax
from jax.experimental import pallas as pl
from jax.experimental.pallas import tpu as pltpu
from jax.experimental.pallas import tpu_sc as plsc

_DIM = 33
_NLUT = 3 * _DIM ** 3
_CSTRIDE = _DIM ** 3
_PLANE = 512 * 512
_B = 16
_PPW = _B * _PLANE // 32
_CHUNK = 1024
_NCHUNK = _PPW // _CHUNK
_GROUPS = _CHUNK // 16
_SCALE = float(_DIM - 1)


def _interp_group(lut_v, xin, xout, o):
    xr = xin[pl.ds(o, 16)]
    xg = xin[pl.ds(_CHUNK + o, 16)]
    xb = xin[pl.ds(2 * _CHUNK + o, 16)]
    r = xr * _SCALE
    g = xg * _SCALE
    b = xb * _SCALE
    r0 = jnp.minimum(r.astype(jnp.int32), _DIM - 2)
    g0 = jnp.minimum(g.astype(jnp.int32), _DIM - 2)
    b0 = jnp.minimum(b.astype(jnp.int32), _DIM - 2)
    fr = r - r0.astype(jnp.float32)
    fg = g - g0.astype(jnp.float32)
    fb = b - b0.astype(jnp.float32)
    base = b0 * (_DIM * _DIM) + g0 * _DIM + r0
    for c in range(3):
        bc = base + c * _CSTRIDE
        v000 = plsc.load_gather(lut_v, [bc])
        v001 = plsc.load_gather(lut_v, [bc + 1])
        v010 = plsc.load_gather(lut_v, [bc + _DIM])
        v011 = plsc.load_gather(lut_v, [bc + (_DIM + 1)])
        v100 = plsc.load_gather(lut_v, [bc + _DIM * _DIM])
        v101 = plsc.load_gather(lut_v, [bc + (_DIM * _DIM + 1)])
        v110 = plsc.load_gather(lut_v, [bc + (_DIM * _DIM + _DIM)])
        v111 = plsc.load_gather(lut_v, [bc + (_DIM * _DIM + _DIM + 1)])
        a00 = v000 + fr * (v001 - v000)
        a01 = v010 + fr * (v011 - v010)
        a10 = v100 + fr * (v101 - v100)
        a11 = v110 + fr * (v111 - v110)
        e0 = a00 + fg * (a01 - a00)
        e1 = a10 + fg * (a11 - a10)
        xout[pl.ds(c * _CHUNK + o, 16)] = e0 + fb * (e1 - e0)


def _body(lut_hbm, x_hbm, out_hbm, lut_v, xin0, xin1, xout0, xout1,
          si0, si1, so0, so1):
    img = lax.axis_index("s")
    half = lax.axis_index("c")
    base = half * _PPW
    pltpu.sync_copy(lut_hbm, lut_v)

    def in_copy(g, buf, sem):
        off = base + g * _CHUNK
        return [
            pltpu.make_async_copy(
                x_hbm.at[pl.ds((img * 3 + c) * _PLANE + off, _CHUNK)],
                buf.at[pl.ds(c * _CHUNK, _CHUNK)],
                sem,
            )
            for c in range(3)
        ]

    def out_copy(g, buf, sem):
        off = base + g * _CHUNK
        return [
            pltpu.make_async_copy(
                buf.at[pl.ds(c * _CHUNK, _CHUNK)],
                out_hbm.at[pl.ds((img * 3 + c) * _PLANE + off, _CHUNK)],
                sem,
            )
            for c in range(3)
        ]

    def compute(xin, xout):
        @plsc.parallel_loop(0, _CHUNK, step=16, unroll=4)
        def _(o):
            _interp_group(lut_v, xin, xout, o)

    for d in in_copy(0, xin0, si0):
        d.start()
    for d in in_copy(1, xin1, si1):
        d.start()

    def pair(i, carry):
        g0 = 2 * i
        for buf_i, (g, xin, xout, si, so) in enumerate(
            ((g0, xin0, xout0, si0, so0), (g0 + 1, xin1, xout1, si1, so1))
        ):
            for d in in_copy(g, xin, si):
                d.wait()

            @pl.when(i > 0)
            def _():
                for d in out_copy(g, xout, so):
                    d.wait()

            compute(xin, xout)
            for d in out_copy(g, xout, so):
                d.start()

            @pl.when(g + 2 < _NCHUNK)
            def _():
                for d in in_copy(g + 2, xin, si):
                    d.start()

        return carry

    lax.fori_loop(0, _NCHUNK // 2, pair, 0)
    for d in out_copy(_NCHUNK - 2, xout0, so0):
        d.wait()
    for d in out_copy(_NCHUNK - 1, xout1, so1):
        d.wait()


@functools.cache
def _build():
    mesh = plsc.VectorSubcoreMesh(
        core_axis_name="c", subcore_axis_name="s", num_cores=2, num_subcores=16
    )
    return pl.kernel(
        _body,
        out_type=jax.ShapeDtypeStruct((_B * 3 * _PLANE,), jnp.float32),
        mesh=mesh,
        scratch_types=[
            pltpu.VMEM((_NLUT,), jnp.float32),
            pltpu.VMEM((3 * _CHUNK,), jnp.float32),
            pltpu.VMEM((3 * _CHUNK,), jnp.float32),
            pltpu.VMEM((3 * _CHUNK,), jnp.float32),
            pltpu.VMEM((3 * _CHUNK,), jnp.float32),
            pltpu.SemaphoreType.DMA,
            pltpu.SemaphoreType.DMA,
            pltpu.SemaphoreType.DMA,
            pltpu.SemaphoreType.DMA,
        ],
        compiler_params=pltpu.CompilerParams(needs_layout_passes=False),
    )


@jax.jit
def kernel(lut, x):
    lut_flat = lut.reshape(_NLUT)
    x_flat = x.reshape(_B * 3 * _PLANE)
    out = _build()(lut_flat, x_flat)
    return out.reshape(_B, 3, 512, 512)
